# baseline (speedup 1.0000x reference)
import functools

import jax
import jax.numpy as jnp
from jax import lax
from jax.experimental import pallas as pl
from jax.experimental.pallas import tpu as pltpu
from jax.experimental.pallas import tpu_sc as plsc

S_SHORT, S, B, D = 2048, 4096, 16, 1024
ROWS = S * B
SRC_ROWS = S_SHORT * B

NC, NS = 2, 16
NW = NC * NS
RPW = ROWS // NW
CHUNK = 32
NCHUNK = RPW // CHUNK
NBUF = 2


def _ln_body(x_ref, w_ref, b_ref, o_ref):
    blk = x_ref.shape[0]
    xb = x_ref[...]
    m = jnp.mean(xb, axis=-1, keepdims=True)
    c = xb - m
    v = jnp.mean(c * c, axis=-1, keepdims=True)
    normed = c * lax.rsqrt(v + 1e-5) * w_ref[...] + b_ref[...]
    u = lax.bitcast_convert_type(normed, jnp.uint32) + jnp.uint32(0x8000)
    u3 = u.reshape(blk, D // 32, 32)
    lo = u3[:, :, :16] >> 16
    hi = u3[:, :, 16:] & jnp.uint32(0xFFFF0000)
    o_ref[...] = (lo | hi).reshape(blk, D // 2)


def _layernorm(x2d, ln_w, ln_b):
    blk = 512
    return pl.pallas_call(
        _ln_body,
        grid=(SRC_ROWS // blk,),
        in_specs=[
            pl.BlockSpec((blk, D), lambda i: (i, 0)),
            pl.BlockSpec((1, D), lambda i: (0, 0)),
            pl.BlockSpec((1, D), lambda i: (0, 0)),
        ],
        out_specs=pl.BlockSpec((blk, D // 2), lambda i: (i, 0)),
        out_shape=jax.ShapeDtypeStruct((SRC_ROWS, D // 2), jnp.uint32),
    )(x2d, ln_w.reshape(1, D), ln_b.reshape(1, D))


def _sc_body(table_hbm, idx_hbm, resid_hbm, out_hbm, idx_v, *scratch):
    bufs_g = scratch[0:NBUF]
    bufs_r = scratch[NBUF : 2 * NBUF]
    sems_g = scratch[2 * NBUF : 3 * NBUF]
    sems_r = scratch[3 * NBUF : 4 * NBUF]
    sems_s = scratch[4 * NBUF : 5 * NBUF]

    wid = lax.axis_index("s") * NC + lax.axis_index("c")
    base = wid * RPW
    pltpu.sync_copy(idx_hbm.at[pl.ds(base, RPW)], idx_v)

    def start_gather(k, b):
        pltpu.async_copy(
            table_hbm.at[idx_v.at[pl.ds(k * CHUNK, CHUNK)]], bufs_g[b], sems_g[b]
        )

    def start_resid(k, b):
        pltpu.async_copy(
            resid_hbm.at[pl.ds(base + k * CHUNK, CHUNK)], bufs_r[b], sems_r[b]
        )

    for b in range(NBUF):
        start_gather(b, b)
        start_resid(b, b)

    @pl.loop(0, NCHUNK, step=NBUF)
    def body(c):
        for b in range(NBUF):
            k = c + b
            row0 = base + k * CHUNK
            pltpu.make_async_copy(
                table_hbm.at[idx_v.at[pl.ds(k * CHUNK, CHUNK)]], bufs_g[b], sems_g[b]
            ).wait()
            pltpu.make_async_copy(
                resid_hbm.at[pl.ds(row0, CHUNK)], bufs_r[b], sems_r[b]
            ).wait()

            @pl.loop(0, CHUNK)
            def add_row(r):
                for i in range(D // 32):
                    v = bufs_g[b][r, pl.ds(i * 16, 16)]
                    lo = lax.bitcast_convert_type(v << jnp.int32(16), jnp.float32)
                    hi = lax.bitcast_convert_type(v & jnp.int32(-65536), jnp.float32)
                    plsc.addupdate(bufs_r[b].at[r, pl.ds(i * 32, 16)], lo)
                    plsc.addupdate(bufs_r[b].at[r, pl.ds(i * 32 + 16, 16)], hi)

            @pl.when(k + NBUF < NCHUNK)
            def _():
                start_gather(k + NBUF, b)

            store = pltpu.async_copy(
                bufs_r[b], out_hbm.at[pl.ds(row0, CHUNK)], sems_s[b]
            )

            @pl.when(k + NBUF < NCHUNK)
            def _():
                store.wait()
                start_resid(k + NBUF, b)

    for b in range(NBUF):
        k_last = NCHUNK - NBUF + b
        pltpu.make_async_copy(
            bufs_r[b], out_hbm.at[pl.ds(base + k_last * CHUNK, CHUNK)], sems_s[b]
        ).wait()


_sc_gather_add = pl.kernel(
    _sc_body,
    out_type=jax.ShapeDtypeStruct((ROWS, D), jnp.float32),
    mesh=plsc.VectorSubcoreMesh(
        core_axis_name="c", subcore_axis_name="s", num_cores=NC, num_subcores=NS
    ),
    scratch_types=(
        [pltpu.VMEM((RPW,), jnp.int32)]
        + [pltpu.VMEM((CHUNK, D // 2), jnp.int32) for _ in range(NBUF)]
        + [pltpu.VMEM((CHUNK, D), jnp.float32) for _ in range(NBUF)]
        + [pltpu.SemaphoreType.DMA for _ in range(3 * NBUF)]
    ),
)


def kernel(x, residual, upsampling_mask, boundaries, ln_w, ln_b):
    del boundaries
    x2d = x.reshape(SRC_ROWS, D)
    normed = _layernorm(x2d, ln_w.astype(jnp.float32), ln_b.astype(jnp.float32))
    normed = lax.bitcast_convert_type(normed, jnp.int32)
    flat_idx = (
        upsampling_mask.T.astype(jnp.int32) * B + jnp.arange(B, dtype=jnp.int32)[None, :]
    ).reshape(ROWS)
    out2d = _sc_gather_add(normed, flat_idx, residual.reshape(ROWS, D))
    return out2d.reshape(S, B, D)


# speedup vs baseline: 1.7069x; 1.7069x over previous
import functools

import jax
import jax.numpy as jnp
from jax import lax
from jax.experimental import pallas as pl
from jax.experimental.pallas import tpu as pltpu
from jax.experimental.pallas import tpu_sc as plsc

S_SHORT, S, B, D = 2048, 4096, 16, 1024
ROWS = S * B
SRC_ROWS = S_SHORT * B

NC, NS = 2, 16
NW = NC * NS
RPW = ROWS // NW
CHUNK = 32
NCHUNK = RPW // CHUNK
NBUF = 2


def _ln_body(x_ref, w_ref, b_ref, o_ref):
    blk = x_ref.shape[0]
    xb = x_ref[...]
    m = jnp.mean(xb, axis=-1, keepdims=True)
    c = xb - m
    v = jnp.mean(c * c, axis=-1, keepdims=True)
    normed = c * lax.rsqrt(v + 1e-5) * w_ref[...] + b_ref[...]
    u = lax.bitcast_convert_type(normed, jnp.uint32) + jnp.uint32(0x8000)
    o_ref[...] = (u[:, : D // 2] >> 16) | (u[:, D // 2 :] & jnp.uint32(0xFFFF0000))


def _layernorm(x2d, ln_w, ln_b):
    blk = 512
    return pl.pallas_call(
        _ln_body,
        grid=(SRC_ROWS // blk,),
        in_specs=[
            pl.BlockSpec((blk, D), lambda i: (i, 0)),
            pl.BlockSpec((1, D), lambda i: (0, 0)),
            pl.BlockSpec((1, D), lambda i: (0, 0)),
        ],
        out_specs=pl.BlockSpec((blk, D // 2), lambda i: (i, 0)),
        out_shape=jax.ShapeDtypeStruct((SRC_ROWS, D // 2), jnp.uint32),
    )(x2d, ln_w.reshape(1, D), ln_b.reshape(1, D))


def _sc_body(table_hbm, idx_hbm, resid_hbm, out_hbm, idx_v, *scratch):
    bufs_g = scratch[0:NBUF]
    bufs_r = scratch[NBUF : 2 * NBUF]
    sems_g = scratch[2 * NBUF : 3 * NBUF]
    sems_r = scratch[3 * NBUF : 4 * NBUF]
    sems_s = scratch[4 * NBUF : 5 * NBUF]

    wid = lax.axis_index("s") * NC + lax.axis_index("c")
    base = wid * RPW
    pltpu.sync_copy(idx_hbm.at[pl.ds(base, RPW)], idx_v)

    def start_gather(k, b):
        pltpu.async_copy(
            table_hbm.at[idx_v.at[pl.ds(k * CHUNK, CHUNK)]], bufs_g[b], sems_g[b]
        )

    def start_resid(k, b):
        pltpu.async_copy(
            resid_hbm.at[pl.ds(base + k * CHUNK, CHUNK)], bufs_r[b], sems_r[b]
        )

    for b in range(NBUF):
        start_gather(b, b)
        start_resid(b, b)

    @pl.loop(0, NCHUNK, step=NBUF)
    def body(c):
        for b in range(NBUF):
            k = c + b
            row0 = base + k * CHUNK
            pltpu.make_async_copy(
                table_hbm.at[idx_v.at[pl.ds(k * CHUNK, CHUNK)]], bufs_g[b], sems_g[b]
            ).wait()
            pltpu.make_async_copy(
                resid_hbm.at[pl.ds(row0, CHUNK)], bufs_r[b], sems_r[b]
            ).wait()

            @pl.loop(0, CHUNK)
            def add_row(r):
                for i in range(D // 32):
                    v = bufs_g[b][r, pl.ds(i * 16, 16)]
                    lo = lax.bitcast_convert_type(v << jnp.int32(16), jnp.float32)
                    hi = lax.bitcast_convert_type(v, jnp.float32)
                    plsc.addupdate(bufs_r[b].at[r, pl.ds(i * 16, 16)], lo)
                    plsc.addupdate(bufs_r[b].at[r, pl.ds(D // 2 + i * 16, 16)], hi)

            @pl.when(k + NBUF < NCHUNK)
            def _():
                start_gather(k + NBUF, b)

            store = pltpu.async_copy(
                bufs_r[b], out_hbm.at[pl.ds(row0, CHUNK)], sems_s[b]
            )

            @pl.when(k + NBUF < NCHUNK)
            def _():
                store.wait()
                start_resid(k + NBUF, b)

    for b in range(NBUF):
        k_last = NCHUNK - NBUF + b
        pltpu.make_async_copy(
            bufs_r[b], out_hbm.at[pl.ds(base + k_last * CHUNK, CHUNK)], sems_s[b]
        ).wait()


_sc_gather_add = pl.kernel(
    _sc_body,
    out_type=jax.ShapeDtypeStruct((ROWS, D), jnp.float32),
    mesh=plsc.VectorSubcoreMesh(
        core_axis_name="c", subcore_axis_name="s", num_cores=NC, num_subcores=NS
    ),
    scratch_types=(
        [pltpu.VMEM((RPW,), jnp.int32)]
        + [pltpu.VMEM((CHUNK, D // 2), jnp.int32) for _ in range(NBUF)]
        + [pltpu.VMEM((CHUNK, D), jnp.float32) for _ in range(NBUF)]
        + [pltpu.SemaphoreType.DMA for _ in range(3 * NBUF)]
    ),
)


def kernel(x, residual, upsampling_mask, boundaries, ln_w, ln_b):
    del boundaries
    x2d = x.reshape(SRC_ROWS, D)
    normed = _layernorm(x2d, ln_w.astype(jnp.float32), ln_b.astype(jnp.float32))
    normed = lax.bitcast_convert_type(normed, jnp.int32)
    flat_idx = (
        upsampling_mask.T.astype(jnp.int32) * B + jnp.arange(B, dtype=jnp.int32)[None, :]
    ).reshape(ROWS)
    out2d = _sc_gather_add(normed, flat_idx, residual.reshape(ROWS, D))
    return out2d.reshape(S, B, D)
